# baseline (speedup 1.0000x reference)
import jax
import jax.numpy as jnp
from jax.experimental import pallas as pl
from jax.experimental.pallas import tpu as pltpu

_B, _N, _K = 64, 196, 4096
_NA = 192
_NT = _N - _NA
_BB = 4
_GRID = _B // _BB


def _dense_kernel(mask_ref, maskt_ref, ps_ref, pst_ref, pt_ref, ptt_ref,
                  num_ref, den_ref):
    part = jnp.float32(0.0)
    cnt = jnp.float32(0.0)
    for b in range(_BB):
        for (mr, sr, tr) in ((mask_ref, ps_ref, pt_ref),
                             (maskt_ref, pst_ref, ptt_ref)):
            ps = sr[b]
            pt = tr[b]
            m = mr[b]
            safe = jnp.where(m > 0.0, ps, jnp.ones_like(ps))
            part += jnp.sum(pt * jnp.log(safe) * m)
            cnt += jnp.sum(m)
    num_ref[...] = (-part).reshape(1, 1, 1)
    den_ref[...] = cnt.reshape(1, 1, 1)


def kernel(ps, pt, bool_masked_pos):
    maskf = bool_masked_pos.astype(jnp.float32)[..., None]
    ps_t = ps[:, _NA:, :]
    pt_t = pt[:, _NA:, :]
    mask_t = maskf[:, _NA:, :]
    main = lambda i: (i, 0, 0)
    num, den = pl.pallas_call(
        _dense_kernel,
        grid=(_GRID,),
        in_specs=[
            pl.BlockSpec((_BB, _NA, 1), main),
            pl.BlockSpec((_BB, _NT, 1), main),
            pl.BlockSpec((_BB, _NA, _K), main),
            pl.BlockSpec((_BB, _NT, _K), main),
            pl.BlockSpec((_BB, _NA, _K), main),
            pl.BlockSpec((_BB, _NT, _K), main),
        ],
        out_specs=[
            pl.BlockSpec((1, 1, 1), lambda i: (i, 0, 0)),
            pl.BlockSpec((1, 1, 1), lambda i: (i, 0, 0)),
        ],
        out_shape=[
            jax.ShapeDtypeStruct((_GRID, 1, 1), jnp.float32),
            jax.ShapeDtypeStruct((_GRID, 1, 1), jnp.float32),
        ],
        compiler_params=pltpu.CompilerParams(
            dimension_semantics=("arbitrary",),
        ),
    )(maskf, mask_t, ps, ps_t, pt, pt_t)
    return jnp.sum(num) / jnp.sum(den)


# speedup vs baseline: 1.0329x; 1.0329x over previous
import jax
import jax.numpy as jnp
from jax.experimental import pallas as pl
from jax.experimental.pallas import tpu as pltpu

_B, _N, _K = 64, 196, 4096
_DEPTH = 6


def _loss_kernel(mask_ref, ps_hbm, pt_hbm, out_ref,
                 ps_buf, pt_buf, ps_sem, pt_sem):
    def _issue(b, slot):
        pltpu.make_async_copy(ps_hbm.at[b], ps_buf.at[slot], ps_sem.at[slot]).start()
        pltpu.make_async_copy(pt_hbm.at[b], pt_buf.at[slot], pt_sem.at[slot]).start()

    for d in range(_DEPTH):
        _issue(d, d)

    def body(b, carry):
        acc = carry
        slot = jax.lax.rem(b, _DEPTH)
        pltpu.make_async_copy(ps_hbm.at[b], ps_buf.at[slot], ps_sem.at[slot]).wait()
        pltpu.make_async_copy(pt_hbm.at[b], pt_buf.at[slot], pt_sem.at[slot]).wait()
        ps = ps_buf[slot]
        pt = pt_buf[slot]
        m = mask_ref[b]
        safe = jnp.where(m > 0.0, ps, jnp.ones_like(ps))
        acc += jnp.sum(pt * jnp.log(safe) * m)

        @pl.when(b + _DEPTH < _B)
        def _():
            _issue(b + _DEPTH, jax.lax.rem(b + _DEPTH, _DEPTH))

        return acc

    num = jax.lax.fori_loop(0, _B, body, jnp.float32(0.0))
    den = jnp.sum(mask_ref[...])
    out_ref[...] = (-num / den).reshape(1, 1)


def kernel(ps, pt, bool_masked_pos):
    maskf = bool_masked_pos.astype(jnp.float32)[..., None]
    out = pl.pallas_call(
        _loss_kernel,
        in_specs=[
            pl.BlockSpec(memory_space=pltpu.VMEM),
            pl.BlockSpec(memory_space=pl.ANY),
            pl.BlockSpec(memory_space=pl.ANY),
        ],
        out_specs=pl.BlockSpec(memory_space=pltpu.VMEM),
        out_shape=jax.ShapeDtypeStruct((1, 1), jnp.float32),
        scratch_shapes=[
            pltpu.VMEM((_DEPTH, _N, _K), jnp.float32),
            pltpu.VMEM((_DEPTH, _N, _K), jnp.float32),
            pltpu.SemaphoreType.DMA((_DEPTH,)),
            pltpu.SemaphoreType.DMA((_DEPTH,)),
        ],
    )(maskf, ps, pt)
    return out[0, 0]
